# baseline (speedup 1.0000x reference)
import functools

import jax
import jax.numpy as jnp
from jax.experimental import pallas as pl
from jax.experimental.pallas import tpu as pltpu

_VMEM_LIMIT = 56 * 1024 * 1024
_BF = jnp.bfloat16


def _c1_kernel(x_ref, adj_ref, w1_ref, s1_ref, adjb_ref):
    adjb_ref[...] = adj_ref[...].astype(_BF)
    s1 = jnp.dot(x_ref[...].astype(_BF), w1_ref[...],
                 preferred_element_type=jnp.float32)
    s1_ref[...] = jnp.tanh(s1).astype(_BF)


def _c2_kernel(adjb_ref, s1_ref, w2_ref, w3_ref, wc1_ref, bc1_ref,
               wc2_ref, bc2_ref, z_ref, c_ref, s2_ref, s3_ref, *, tm):
    p = pl.program_id(0)
    k = pl.program_id(1)
    rows = pl.ds(k * tm, tm)
    adjb = adjb_ref[...]

    @pl.when(p == 0)
    def _phase0():
        z1 = jnp.dot(adjb, s1_ref[...], preferred_element_type=jnp.float32)
        s2 = jnp.dot(z1.astype(_BF), w2_ref[...],
                     preferred_element_type=jnp.float32)
        s2_ref[rows, :] = jnp.tanh(s2).astype(_BF)

    @pl.when(p == 1)
    def _phase1():
        z2 = jnp.dot(adjb, s2_ref[...], preferred_element_type=jnp.float32)
        s3 = jnp.dot(z2.astype(_BF), w3_ref[...],
                     preferred_element_type=jnp.float32)
        s3_ref[rows, :] = s3.astype(_BF)

    @pl.when(p == 2)
    def _phase2():
        z = jnp.dot(adjb, s3_ref[...], preferred_element_type=jnp.float32)
        z_ref[...] = z
        h = jnp.dot(z.astype(_BF), wc1_ref[...],
                    preferred_element_type=jnp.float32) + bc1_ref[...]
        h = jnp.maximum(h, 0.0)
        logits = jnp.dot(h.astype(_BF), wc2_ref[...],
                         preferred_element_type=jnp.float32) + bc2_ref[...]
        m = jnp.max(logits, axis=-1, keepdims=True)
        e = jnp.exp(logits - m)
        c_ref[...] = e * pl.reciprocal(jnp.sum(e, axis=-1, keepdims=True))


def _row_spec(tm, d):
    return pl.BlockSpec((tm, d), lambda i: (i, 0))


def _full_spec1(shape):
    return pl.BlockSpec(shape, lambda i, _s=shape: tuple(0 for _ in _s))


def _full_spec2(shape):
    return pl.BlockSpec(shape, lambda p, k, _s=shape: tuple(0 for _ in _s))


def kernel(x, adj, w1, w2, w3, wc1, bc1, wc2, bc2):
    N, n_input = x.shape
    enc1, enc2, enc3 = w1.shape[1], w2.shape[1], w3.shape[1]
    nc = wc2.shape[1]
    tm = min(512, N)
    G = pl.cdiv(N, tm)
    cp1 = pltpu.CompilerParams(dimension_semantics=("arbitrary",),
                               vmem_limit_bytes=_VMEM_LIMIT)
    cp2 = pltpu.CompilerParams(dimension_semantics=("arbitrary", "arbitrary"),
                               vmem_limit_bytes=_VMEM_LIMIT)

    w1b = w1.astype(_BF)
    w2b = w2.astype(_BF)
    w3b = w3.astype(_BF)
    wc1b = wc1.astype(_BF)
    wc2b = wc2.astype(_BF)

    s1, adjb = pl.pallas_call(
        _c1_kernel,
        out_shape=(jax.ShapeDtypeStruct((N, enc1), _BF),
                   jax.ShapeDtypeStruct((N, N), _BF)),
        grid=(G,),
        in_specs=[_row_spec(tm, n_input), _row_spec(tm, N),
                  _full_spec1((n_input, enc1))],
        out_specs=(_row_spec(tm, enc1), _row_spec(tm, N)),
        compiler_params=cp1,
    )(x, adj, w1b)

    z_igae, c = pl.pallas_call(
        functools.partial(_c2_kernel, tm=tm),
        out_shape=(jax.ShapeDtypeStruct((N, enc3), jnp.float32),
                   jax.ShapeDtypeStruct((N, nc), jnp.float32)),
        grid=(3, G),
        in_specs=[pl.BlockSpec((tm, N), lambda p, k: (k, 0)),
                  _full_spec2((N, enc1)),
                  _full_spec2((enc1, enc2)), _full_spec2((enc2, enc3)),
                  _full_spec2((enc3, enc3)), _full_spec2((1, enc3)),
                  _full_spec2((enc3, nc)), _full_spec2((1, nc))],
        out_specs=(pl.BlockSpec((tm, enc3), lambda p, k: (k * (p == 2), 0)),
                   pl.BlockSpec((tm, nc), lambda p, k: (k * (p == 2), 0))),
        scratch_shapes=[pltpu.VMEM((N, enc2), _BF),
                        pltpu.VMEM((N, enc3), _BF)],
        compiler_params=cp2,
    )(adjb, s1, w2b, w3b, wc1b, bc1, wc2b, bc2)

    return z_igae, c


# speedup vs baseline: 1.0848x; 1.0848x over previous
import functools

import jax
import jax.numpy as jnp
from jax.experimental import pallas as pl
from jax.experimental.pallas import tpu as pltpu

_VMEM_LIMIT = 56 * 1024 * 1024
_BF = jnp.bfloat16
_F8 = jnp.float8_e4m3fn
_ADJ_SCALE = 256.0
_INV_ADJ_SCALE = 1.0 / 256.0


def _c1_kernel(x_ref, adj_ref, w1_ref, s1_ref, adjb_ref):
    adjb_ref[...] = (adj_ref[...] * _ADJ_SCALE).astype(_F8)
    s1 = jnp.dot(x_ref[...].astype(_BF), w1_ref[...],
                 preferred_element_type=jnp.float32)
    s1_ref[...] = jnp.tanh(s1).astype(_BF)


def _c2_kernel(adjb_ref, s1_ref, w2_ref, w3_ref, wc1_ref, bc1_ref,
               wc2_ref, bc2_ref, z_ref, c_ref, s2_ref, s3_ref, *, tm):
    p = pl.program_id(0)
    k = pl.program_id(1)
    rows = pl.ds(k * tm, tm)
    adjb = adjb_ref[...].astype(_BF)

    @pl.when(p == 0)
    def _phase0():
        z1 = jnp.dot(adjb, s1_ref[...], preferred_element_type=jnp.float32)
        s2 = jnp.dot(z1.astype(_BF), w2_ref[...],
                     preferred_element_type=jnp.float32)
        s2_ref[rows, :] = jnp.tanh(s2).astype(_BF)

    @pl.when(p == 1)
    def _phase1():
        z2 = jnp.dot(adjb, s2_ref[...], preferred_element_type=jnp.float32)
        s3 = jnp.dot(z2.astype(_BF), w3_ref[...],
                     preferred_element_type=jnp.float32)
        s3_ref[rows, :] = s3.astype(_BF)

    @pl.when(p == 2)
    def _phase2():
        z = jnp.dot(adjb, s3_ref[...],
                    preferred_element_type=jnp.float32) * _INV_ADJ_SCALE
        z_ref[...] = z
        h = jnp.dot(z.astype(_BF), wc1_ref[...],
                    preferred_element_type=jnp.float32) + bc1_ref[...]
        h = jnp.maximum(h, 0.0)
        logits = jnp.dot(h.astype(_BF), wc2_ref[...],
                         preferred_element_type=jnp.float32) + bc2_ref[...]
        m = jnp.max(logits, axis=-1, keepdims=True)
        e = jnp.exp(logits - m)
        c_ref[...] = e * pl.reciprocal(jnp.sum(e, axis=-1, keepdims=True))


def _row_spec(tm, d):
    return pl.BlockSpec((tm, d), lambda i: (i, 0))


def _full_spec1(shape):
    return pl.BlockSpec(shape, lambda i, _s=shape: tuple(0 for _ in _s))


def _full_spec2(shape):
    return pl.BlockSpec(shape, lambda p, k, _s=shape: tuple(0 for _ in _s))


def kernel(x, adj, w1, w2, w3, wc1, bc1, wc2, bc2):
    N, n_input = x.shape
    enc1, enc2, enc3 = w1.shape[1], w2.shape[1], w3.shape[1]
    nc = wc2.shape[1]
    tm = min(512, N)
    G = pl.cdiv(N, tm)
    cp1 = pltpu.CompilerParams(dimension_semantics=("arbitrary",),
                               vmem_limit_bytes=_VMEM_LIMIT)
    cp2 = pltpu.CompilerParams(dimension_semantics=("arbitrary", "arbitrary"),
                               vmem_limit_bytes=_VMEM_LIMIT)

    w1b = w1.astype(_BF)
    w2b = (w2 * _INV_ADJ_SCALE).astype(_BF)
    w3b = (w3 * _INV_ADJ_SCALE).astype(_BF)
    wc1b = wc1.astype(_BF)
    wc2b = wc2.astype(_BF)

    s1, adjb = pl.pallas_call(
        _c1_kernel,
        out_shape=(jax.ShapeDtypeStruct((N, enc1), _BF),
                   jax.ShapeDtypeStruct((N, N), _F8)),
        grid=(G,),
        in_specs=[_row_spec(tm, n_input), _row_spec(tm, N),
                  _full_spec1((n_input, enc1))],
        out_specs=(_row_spec(tm, enc1), _row_spec(tm, N)),
        compiler_params=cp1,
    )(x, adj, w1b)

    z_igae, c = pl.pallas_call(
        functools.partial(_c2_kernel, tm=tm),
        out_shape=(jax.ShapeDtypeStruct((N, enc3), jnp.float32),
                   jax.ShapeDtypeStruct((N, nc), jnp.float32)),
        grid=(3, G),
        in_specs=[pl.BlockSpec((tm, N), lambda p, k: (k, 0)),
                  _full_spec2((N, enc1)),
                  _full_spec2((enc1, enc2)), _full_spec2((enc2, enc3)),
                  _full_spec2((enc3, enc3)), _full_spec2((1, enc3)),
                  _full_spec2((enc3, nc)), _full_spec2((1, nc))],
        out_specs=(pl.BlockSpec((tm, enc3), lambda p, k: (k * (p == 2), 0)),
                   pl.BlockSpec((tm, nc), lambda p, k: (k * (p == 2), 0))),
        scratch_shapes=[pltpu.VMEM((N, enc2), _BF),
                        pltpu.VMEM((N, enc3), _BF)],
        compiler_params=cp2,
    )(adjb, s1, w2b, w3b, wc1b, bc1, wc2b, bc2)

    return z_igae, c


# speedup vs baseline: 1.2011x; 1.1072x over previous
import functools

import jax
import jax.numpy as jnp
from jax.experimental import pallas as pl
from jax.experimental.pallas import tpu as pltpu

_VMEM_LIMIT = 56 * 1024 * 1024
_BF = jnp.bfloat16
_F8 = jnp.float8_e4m3fn
_ADJ_SCALE = 256.0
_INV_ADJ_SCALE = 1.0 / 256.0


def _cast_kernel(adj_ref, adjb_ref):
    adjb_ref[...] = (adj_ref[...] * _ADJ_SCALE).astype(_F8)


def _gaec_kernel(adjb_ref, x_ref, w1_ref, w2_ref, w3_ref, wc1_ref, bc1_ref,
                 wc2_ref, bc2_ref, z_ref, c_ref, s1_ref, s2_ref, s3_ref,
                 *, tm):
    p = pl.program_id(0)
    k = pl.program_id(1)
    rows = pl.ds(k * tm, tm)

    @pl.when(p == 0)
    def _phase0():
        s1 = jnp.dot(x_ref[...].astype(_BF), w1_ref[...].astype(_BF),
                     preferred_element_type=jnp.float32)
        s1_ref[rows, :] = jnp.tanh(s1).astype(_BF)

    @pl.when(p == 1)
    def _phase1():
        adjb = adjb_ref[...].astype(_BF)
        z1 = jnp.dot(adjb, s1_ref[...], preferred_element_type=jnp.float32)
        w2 = (w2_ref[...] * _INV_ADJ_SCALE).astype(_BF)
        s2 = jnp.dot(z1.astype(_BF), w2, preferred_element_type=jnp.float32)
        s2_ref[rows, :] = jnp.tanh(s2).astype(_BF)

    @pl.when(p == 2)
    def _phase2():
        adjb = adjb_ref[...].astype(_BF)
        z2 = jnp.dot(adjb, s2_ref[...], preferred_element_type=jnp.float32)
        w3 = (w3_ref[...] * _INV_ADJ_SCALE).astype(_BF)
        s3 = jnp.dot(z2.astype(_BF), w3, preferred_element_type=jnp.float32)
        s3_ref[rows, :] = s3.astype(_BF)

    @pl.when(p == 3)
    def _phase3():
        adjb = adjb_ref[...].astype(_BF)
        z = jnp.dot(adjb, s3_ref[...],
                    preferred_element_type=jnp.float32) * _INV_ADJ_SCALE
        z_ref[...] = z
        h = jnp.dot(z.astype(_BF), wc1_ref[...].astype(_BF),
                    preferred_element_type=jnp.float32) + bc1_ref[...]
        h = jnp.maximum(h, 0.0)
        logits = jnp.dot(h.astype(_BF), wc2_ref[...].astype(_BF),
                         preferred_element_type=jnp.float32) + bc2_ref[...]
        m = jnp.max(logits, axis=-1, keepdims=True)
        e = jnp.exp(logits - m)
        c_ref[...] = e * pl.reciprocal(jnp.sum(e, axis=-1, keepdims=True))


def _row_spec(tm, d):
    return pl.BlockSpec((tm, d), lambda i: (i, 0))


def _full_spec(shape):
    return pl.BlockSpec(shape, lambda p, k, _s=shape: tuple(0 for _ in _s))


def kernel(x, adj, w1, w2, w3, wc1, bc1, wc2, bc2):
    N, n_input = x.shape
    enc1, enc2, enc3 = w1.shape[1], w2.shape[1], w3.shape[1]
    nc = wc2.shape[1]
    tm = min(512, N)
    G = pl.cdiv(N, tm)
    last = G - 1

    adjb = pl.pallas_call(
        _cast_kernel,
        out_shape=jax.ShapeDtypeStruct((N, N), _F8),
        grid=(G,),
        in_specs=[_row_spec(tm, N)],
        out_specs=_row_spec(tm, N),
        compiler_params=pltpu.CompilerParams(
            dimension_semantics=("arbitrary",),
            vmem_limit_bytes=_VMEM_LIMIT),
    )(adj)

    z_igae, c = pl.pallas_call(
        functools.partial(_gaec_kernel, tm=tm),
        out_shape=(jax.ShapeDtypeStruct((N, enc3), jnp.float32),
                   jax.ShapeDtypeStruct((N, nc), jnp.float32)),
        grid=(4, G),
        in_specs=[
            pl.BlockSpec((tm, N),
                         lambda p, k: (k * (p >= 1) + last * (p == 0), 0)),
            pl.BlockSpec((tm, n_input),
                         lambda p, k: (k * (p == 0) + last * (p >= 1), 0)),
            _full_spec((n_input, enc1)), _full_spec((enc1, enc2)),
            _full_spec((enc2, enc3)), _full_spec((enc3, enc3)),
            _full_spec((1, enc3)), _full_spec((enc3, nc)),
            _full_spec((1, nc)),
        ],
        out_specs=(pl.BlockSpec((tm, enc3), lambda p, k: (k * (p == 3), 0)),
                   pl.BlockSpec((tm, nc), lambda p, k: (k * (p == 3), 0))),
        scratch_shapes=[pltpu.VMEM((N, enc1), _BF),
                        pltpu.VMEM((N, enc2), _BF),
                        pltpu.VMEM((N, enc3), _BF)],
        compiler_params=pltpu.CompilerParams(
            dimension_semantics=("arbitrary", "arbitrary"),
            vmem_limit_bytes=_VMEM_LIMIT),
    )(adjb, x, w1, w2, w3, wc1, bc1, wc2, bc2)

    return z_igae, c


# speedup vs baseline: 1.3057x; 1.0871x over previous
import functools

import jax
import jax.numpy as jnp
from jax.experimental import pallas as pl
from jax.experimental.pallas import tpu as pltpu

_VMEM_LIMIT = 56 * 1024 * 1024
_BF = jnp.bfloat16
_F8 = jnp.float8_e4m3fn
_ADJ_SCALE = 256.0
_INV_ADJ_SCALE = 1.0 / 256.0


def _cast_kernel(adj_ref, adjb_ref):
    adjb_ref[...] = (adj_ref[...] * _ADJ_SCALE).astype(_F8)


def _gaec_kernel(adjb_ref, x_ref, w1_ref, w2_ref, w3_ref, wc1_ref, bc1_ref,
                 wc2_ref, bc2_ref, z_ref, c_ref, s1_ref, s2_ref, s3_ref,
                 *, tm):
    p = pl.program_id(0)
    k = pl.program_id(1)
    rows = pl.ds(k * tm, tm)

    @pl.when(p == 0)
    def _phase0():
        s1 = jnp.dot(x_ref[...].astype(_BF), w1_ref[...].astype(_BF),
                     preferred_element_type=jnp.float32)
        s1_ref[rows, :] = jnp.tanh(s1).astype(_BF)

    @pl.when(p == 1)
    def _phase1():
        adjb = adjb_ref[...].astype(_BF)
        z1 = jnp.dot(adjb, s1_ref[...], preferred_element_type=jnp.float32)
        w2 = (w2_ref[...] * _INV_ADJ_SCALE).astype(_BF)
        s2 = jnp.dot(z1.astype(_BF), w2, preferred_element_type=jnp.float32)
        s2_ref[rows, :] = jnp.tanh(s2).astype(_BF)

    @pl.when(p == 2)
    def _phase2():
        adjb = adjb_ref[...].astype(_BF)
        z2 = jnp.dot(adjb, s2_ref[...], preferred_element_type=jnp.float32)
        w3 = (w3_ref[...] * _INV_ADJ_SCALE).astype(_BF)
        s3 = jnp.dot(z2.astype(_BF), w3, preferred_element_type=jnp.float32)
        s3_ref[rows, :] = s3.astype(_BF)

    @pl.when(p == 3)
    def _phase3():
        adjb = adjb_ref[...].astype(_BF)
        z = jnp.dot(adjb, s3_ref[...],
                    preferred_element_type=jnp.float32) * _INV_ADJ_SCALE
        z_ref[...] = z
        h = jnp.dot(z.astype(_BF), wc1_ref[...].astype(_BF),
                    preferred_element_type=jnp.float32) + bc1_ref[...]
        h = jnp.maximum(h, 0.0)
        logits = jnp.dot(h.astype(_BF), wc2_ref[...].astype(_BF),
                         preferred_element_type=jnp.float32) + bc2_ref[...]
        m = jnp.max(logits, axis=-1, keepdims=True)
        e = jnp.exp(logits - m)
        c_ref[...] = e * pl.reciprocal(jnp.sum(e, axis=-1, keepdims=True))


def _row_spec(tm, d):
    return pl.BlockSpec((tm, d), lambda i: (i, 0))


def _full_spec(shape):
    return pl.BlockSpec(shape, lambda p, k, _s=shape: tuple(0 for _ in _s))


def kernel(x, adj, w1, w2, w3, wc1, bc1, wc2, bc2):
    N, n_input = x.shape
    enc1, enc2, enc3 = w1.shape[1], w2.shape[1], w3.shape[1]
    nc = wc2.shape[1]
    tm = min(512, N)
    G = pl.cdiv(N, tm)
    tm2 = min(1024, N)
    G2 = pl.cdiv(N, tm2)
    last = G2 - 1

    adjb = pl.pallas_call(
        _cast_kernel,
        out_shape=jax.ShapeDtypeStruct((N, N), _F8),
        grid=(G,),
        in_specs=[_row_spec(tm, N)],
        out_specs=_row_spec(tm, N),
        compiler_params=pltpu.CompilerParams(
            dimension_semantics=("arbitrary",),
            vmem_limit_bytes=_VMEM_LIMIT),
    )(adj)

    z_igae, c = pl.pallas_call(
        functools.partial(_gaec_kernel, tm=tm2),
        out_shape=(jax.ShapeDtypeStruct((N, enc3), jnp.float32),
                   jax.ShapeDtypeStruct((N, nc), jnp.float32)),
        grid=(4, G2),
        in_specs=[
            pl.BlockSpec((tm2, N),
                         lambda p, k: (k * (p >= 1) + last * (p == 0), 0)),
            pl.BlockSpec((tm2, n_input),
                         lambda p, k: (k * (p == 0) + last * (p >= 1), 0)),
            _full_spec((n_input, enc1)), _full_spec((enc1, enc2)),
            _full_spec((enc2, enc3)), _full_spec((enc3, enc3)),
            _full_spec((1, enc3)), _full_spec((enc3, nc)),
            _full_spec((1, nc)),
        ],
        out_specs=(pl.BlockSpec((tm2, enc3), lambda p, k: (k * (p == 3), 0)),
                   pl.BlockSpec((tm2, nc), lambda p, k: (k * (p == 3), 0))),
        scratch_shapes=[pltpu.VMEM((N, enc1), _BF),
                        pltpu.VMEM((N, enc2), _BF),
                        pltpu.VMEM((N, enc3), _BF)],
        compiler_params=pltpu.CompilerParams(
            dimension_semantics=("arbitrary", "arbitrary"),
            vmem_limit_bytes=_VMEM_LIMIT),
    )(adjb, x, w1, w2, w3, wc1, bc1, wc2, bc2)

    return z_igae, c


# speedup vs baseline: 1.3156x; 1.0076x over previous
import functools

import jax
import jax.numpy as jnp
from jax.experimental import pallas as pl
from jax.experimental.pallas import tpu as pltpu

_VMEM_LIMIT = 56 * 1024 * 1024
_BF = jnp.bfloat16
_F8 = jnp.float8_e4m3fn
_ADJ_SCALE = 256.0
_INV_ADJ_SCALE = 1.0 / 256.0


def _cast_kernel(adj_ref, adjb_ref):
    adjb_ref[...] = (adj_ref[...] * _ADJ_SCALE).astype(_F8)


def _gaec_kernel(adjb_ref, x_ref, w1_ref, w2_ref, w3_ref, wc1_ref, bc1_ref,
                 wc2_ref, bc2_ref, z_ref, c_ref, s1_ref, s2_ref, s3_ref,
                 *, tm):
    p = pl.program_id(0)
    k = pl.program_id(1)
    rows = pl.ds(k * tm, tm)

    @pl.when(p == 0)
    def _phase0():
        s1 = jnp.dot(x_ref[...].astype(_BF), w1_ref[...].astype(_BF),
                     preferred_element_type=jnp.float32)
        s1_ref[rows, :] = jnp.tanh(s1).astype(_BF)

    @pl.when(p == 1)
    def _phase1():
        adjb = adjb_ref[...].astype(_BF)
        z1 = jnp.dot(adjb, s1_ref[...], preferred_element_type=jnp.float32)
        w2 = (w2_ref[...] * _INV_ADJ_SCALE).astype(_BF)
        s2 = jnp.dot(z1.astype(_BF), w2, preferred_element_type=jnp.float32)
        s2_ref[rows, :] = jnp.tanh(s2).astype(_BF)

    @pl.when(p == 2)
    def _phase2():
        adjb = adjb_ref[...].astype(_BF)
        z2 = jnp.dot(adjb, s2_ref[...], preferred_element_type=jnp.float32)
        w3 = (w3_ref[...] * _INV_ADJ_SCALE).astype(_BF)
        s3 = jnp.dot(z2.astype(_BF), w3, preferred_element_type=jnp.float32)
        s3_ref[rows, :] = s3.astype(_BF)

    @pl.when(p == 3)
    def _phase3():
        adjb = adjb_ref[...].astype(_BF)
        z = jnp.dot(adjb, s3_ref[...],
                    preferred_element_type=jnp.float32) * _INV_ADJ_SCALE
        z_ref[...] = z
        h = jnp.dot(z.astype(_BF), wc1_ref[...].astype(_BF),
                    preferred_element_type=jnp.float32) + bc1_ref[...]
        h = jnp.maximum(h, 0.0)
        logits = jnp.dot(h.astype(_BF), wc2_ref[...].astype(_BF),
                         preferred_element_type=jnp.float32) + bc2_ref[...]
        m = jnp.max(logits, axis=-1, keepdims=True)
        e = jnp.exp(logits - m)
        c_ref[...] = e * pl.reciprocal(jnp.sum(e, axis=-1, keepdims=True))


def _row_spec(tm, d):
    return pl.BlockSpec((tm, d), lambda i: (i, 0))


def _full_spec(shape):
    return pl.BlockSpec(shape, lambda p, k, _s=shape: tuple(0 for _ in _s))


def kernel(x, adj, w1, w2, w3, wc1, bc1, wc2, bc2):
    N, n_input = x.shape
    enc1, enc2, enc3 = w1.shape[1], w2.shape[1], w3.shape[1]
    nc = wc2.shape[1]
    tm = min(512, N)
    G = pl.cdiv(N, tm)
    tm2 = min(2048, N)
    G2 = pl.cdiv(N, tm2)
    last = G2 - 1

    adjb = pl.pallas_call(
        _cast_kernel,
        out_shape=jax.ShapeDtypeStruct((N, N), _F8),
        grid=(G,),
        in_specs=[_row_spec(tm, N)],
        out_specs=_row_spec(tm, N),
        compiler_params=pltpu.CompilerParams(
            dimension_semantics=("arbitrary",),
            vmem_limit_bytes=_VMEM_LIMIT),
    )(adj)

    z_igae, c = pl.pallas_call(
        functools.partial(_gaec_kernel, tm=tm2),
        out_shape=(jax.ShapeDtypeStruct((N, enc3), jnp.float32),
                   jax.ShapeDtypeStruct((N, nc), jnp.float32)),
        grid=(4, G2),
        in_specs=[
            pl.BlockSpec((tm2, N),
                         lambda p, k: (k * (p >= 1) + last * (p == 0), 0)),
            pl.BlockSpec((tm2, n_input),
                         lambda p, k: (k * (p == 0) + last * (p >= 1), 0)),
            _full_spec((n_input, enc1)), _full_spec((enc1, enc2)),
            _full_spec((enc2, enc3)), _full_spec((enc3, enc3)),
            _full_spec((1, enc3)), _full_spec((enc3, nc)),
            _full_spec((1, nc)),
        ],
        out_specs=(pl.BlockSpec((tm2, enc3), lambda p, k: (k * (p == 3), 0)),
                   pl.BlockSpec((tm2, nc), lambda p, k: (k * (p == 3), 0))),
        scratch_shapes=[pltpu.VMEM((N, enc1), _BF),
                        pltpu.VMEM((N, enc2), _BF),
                        pltpu.VMEM((N, enc3), _BF)],
        compiler_params=pltpu.CompilerParams(
            dimension_semantics=("arbitrary", "arbitrary"),
            vmem_limit_bytes=_VMEM_LIMIT),
    )(adjb, x, w1, w2, w3, wc1, bc1, wc2, bc2)

    return z_igae, c


# speedup vs baseline: 1.4196x; 1.0790x over previous
import functools

import jax
import jax.numpy as jnp
from jax.experimental import pallas as pl
from jax.experimental.pallas import tpu as pltpu

_VMEM_LIMIT = 56 * 1024 * 1024
_BF = jnp.bfloat16
_F8 = jnp.float8_e4m3fn
_ADJ_SCALE = 256.0
_INV_ADJ_SCALE = 1.0 / 256.0


def _front_kernel(x_ref, adj_ref, w1_ref, w2_ref, s2_ref, adjb_ref,
                  s1_ref, *, tm):
    p = pl.program_id(0)
    k = pl.program_id(1)
    rows = pl.ds(k * tm, tm)

    @pl.when(p == 0)
    def _phase_a():
        s1 = jnp.dot(x_ref[...].astype(_BF), w1_ref[...].astype(_BF),
                     preferred_element_type=jnp.float32)
        s1_ref[rows, :] = jnp.tanh(s1).astype(_BF)

    @pl.when(p == 1)
    def _phase_b():
        adjf = adj_ref[...]
        adjb_ref[...] = (adjf * _ADJ_SCALE).astype(_F8)
        z1 = jnp.dot(adjf.astype(_BF), s1_ref[...],
                     preferred_element_type=jnp.float32)
        s2 = jnp.dot(z1.astype(_BF), w2_ref[...].astype(_BF),
                     preferred_element_type=jnp.float32)
        s2_ref[...] = jnp.tanh(s2).astype(_BF)


def _back_kernel(adjb_ref, s2_ref, w3_ref, wc1_ref, bc1_ref, wc2_ref,
                 bc2_ref, z_ref, c_ref, s3_ref, *, tm):
    p = pl.program_id(0)
    k = pl.program_id(1)
    rows = pl.ds(k * tm, tm)
    adjb = adjb_ref[...].astype(_BF)

    @pl.when(p == 0)
    def _phase_c():
        z2 = jnp.dot(adjb, s2_ref[...], preferred_element_type=jnp.float32)
        w3 = (w3_ref[...] * _INV_ADJ_SCALE).astype(_BF)
        s3 = jnp.dot(z2.astype(_BF), w3, preferred_element_type=jnp.float32)
        s3_ref[rows, :] = s3.astype(_BF)

    @pl.when(p == 1)
    def _phase_d():
        z = jnp.dot(adjb, s3_ref[...],
                    preferred_element_type=jnp.float32) * _INV_ADJ_SCALE
        z_ref[...] = z
        h = jnp.dot(z.astype(_BF), wc1_ref[...].astype(_BF),
                    preferred_element_type=jnp.float32) + bc1_ref[...]
        h = jnp.maximum(h, 0.0)
        logits = jnp.dot(h.astype(_BF), wc2_ref[...].astype(_BF),
                         preferred_element_type=jnp.float32) + bc2_ref[...]
        m = jnp.max(logits, axis=-1, keepdims=True)
        e = jnp.exp(logits - m)
        c_ref[...] = e * pl.reciprocal(jnp.sum(e, axis=-1, keepdims=True))


def _full_spec(shape):
    return pl.BlockSpec(shape, lambda p, k, _s=shape: tuple(0 for _ in _s))


def kernel(x, adj, w1, w2, w3, wc1, bc1, wc2, bc2):
    N, n_input = x.shape
    enc1, enc2, enc3 = w1.shape[1], w2.shape[1], w3.shape[1]
    nc = wc2.shape[1]
    tm = min(512, N)
    G = pl.cdiv(N, tm)
    last = G - 1
    tm2 = min(2048, N)
    G2 = pl.cdiv(N, tm2)

    s2, adjb = pl.pallas_call(
        functools.partial(_front_kernel, tm=tm),
        out_shape=(jax.ShapeDtypeStruct((N, enc2), _BF),
                   jax.ShapeDtypeStruct((N, N), _F8)),
        grid=(2, G),
        in_specs=[
            pl.BlockSpec((tm, n_input),
                         lambda p, k: (k * (p == 0) + last * (p == 1), 0)),
            pl.BlockSpec((tm, N),
                         lambda p, k: (k * (p == 1) + last * (p == 0), 0)),
            _full_spec((n_input, enc1)), _full_spec((enc1, enc2)),
        ],
        out_specs=(pl.BlockSpec((tm, enc2), lambda p, k: (k * (p == 1), 0)),
                   pl.BlockSpec((tm, N), lambda p, k: (k * (p == 1), 0))),
        scratch_shapes=[pltpu.VMEM((N, enc1), _BF)],
        compiler_params=pltpu.CompilerParams(
            dimension_semantics=("arbitrary", "arbitrary"),
            vmem_limit_bytes=_VMEM_LIMIT),
    )(x, adj, w1, w2)

    z_igae, c = pl.pallas_call(
        functools.partial(_back_kernel, tm=tm2),
        out_shape=(jax.ShapeDtypeStruct((N, enc3), jnp.float32),
                   jax.ShapeDtypeStruct((N, nc), jnp.float32)),
        grid=(2, G2),
        in_specs=[
            pl.BlockSpec((tm2, N), lambda p, k: (k, 0)),
            _full_spec((N, enc2)), _full_spec((enc2, enc3)),
            _full_spec((enc3, enc3)), _full_spec((1, enc3)),
            _full_spec((enc3, nc)), _full_spec((1, nc)),
        ],
        out_specs=(pl.BlockSpec((tm2, enc3), lambda p, k: (k * (p == 1), 0)),
                   pl.BlockSpec((tm2, nc), lambda p, k: (k * (p == 1), 0))),
        scratch_shapes=[pltpu.VMEM((N, enc3), _BF)],
        compiler_params=pltpu.CompilerParams(
            dimension_semantics=("arbitrary", "arbitrary"),
            vmem_limit_bytes=_VMEM_LIMIT),
    )(adjb, s2, w3, wc1, bc1, wc2, bc2)

    return z_igae, c


# speedup vs baseline: 1.4485x; 1.0204x over previous
import functools

import jax
import jax.numpy as jnp
from jax.experimental import pallas as pl
from jax.experimental.pallas import tpu as pltpu

_VMEM_LIMIT = 56 * 1024 * 1024
_BF = jnp.bfloat16
_F8 = jnp.float8_e4m3fn
_ADJ_SCALE = 256.0
_INV_ADJ_SCALE = 1.0 / 256.0


def _front_kernel(x_ref, adj_ref, w1_ref, w2_ref, s2_ref, adjb_ref,
                  s1_ref, *, tm):
    p = pl.program_id(0)
    k = pl.program_id(1)
    rows = pl.ds(k * tm, tm)

    @pl.when(p == 0)
    def _phase_a():
        s1 = jnp.dot(x_ref[...].astype(_BF), w1_ref[...].astype(_BF),
                     preferred_element_type=jnp.float32)
        s1_ref[rows, :] = jnp.tanh(s1).astype(_BF)

    @pl.when(p == 1)
    def _phase_b():
        adjf = adj_ref[...]
        adjb_ref[...] = (adjf * _ADJ_SCALE).astype(_F8)
        z1 = jnp.dot(adjf.astype(_BF), s1_ref[...],
                     preferred_element_type=jnp.float32)
        s2 = jnp.dot(z1.astype(_BF), w2_ref[...].astype(_BF),
                     preferred_element_type=jnp.float32)
        s2_ref[...] = jnp.tanh(s2).astype(_BF)


def _back_kernel(adjb_ref, s2_ref, w3_ref, wc1_ref, bc1_ref, wc2_ref,
                 bc2_ref, z_ref, c_ref, s3_ref, *, tm):
    p = pl.program_id(0)
    k = pl.program_id(1)
    rows = pl.ds(k * tm, tm)
    adjb = adjb_ref[...].astype(_BF)

    @pl.when(p == 0)
    def _phase_c():
        z2 = jnp.dot(adjb, s2_ref[...], preferred_element_type=jnp.float32)
        w3 = (w3_ref[...] * _INV_ADJ_SCALE).astype(_BF)
        s3 = jnp.dot(z2.astype(_BF), w3, preferred_element_type=jnp.float32)
        s3_ref[rows, :] = s3.astype(_BF)

    @pl.when(p == 1)
    def _phase_d():
        z = jnp.dot(adjb, s3_ref[...],
                    preferred_element_type=jnp.float32) * _INV_ADJ_SCALE
        z_ref[...] = z
        h = jnp.dot(z.astype(_BF), wc1_ref[...].astype(_BF),
                    preferred_element_type=jnp.float32) + bc1_ref[...]
        h = jnp.maximum(h, 0.0)
        logits = jnp.dot(h.astype(_BF), wc2_ref[...].astype(_BF),
                         preferred_element_type=jnp.float32) + bc2_ref[...]
        m = jnp.max(logits, axis=-1, keepdims=True)
        e = jnp.exp(logits - m)
        c_ref[...] = e * pl.reciprocal(jnp.sum(e, axis=-1, keepdims=True))


def _full_spec(shape):
    return pl.BlockSpec(shape, lambda p, k, _s=shape: tuple(0 for _ in _s))


def kernel(x, adj, w1, w2, w3, wc1, bc1, wc2, bc2):
    N, n_input = x.shape
    enc1, enc2, enc3 = w1.shape[1], w2.shape[1], w3.shape[1]
    nc = wc2.shape[1]
    tm = min(1024, N)
    G = pl.cdiv(N, tm)
    last = G - 1
    tm2 = min(2048, N)
    G2 = pl.cdiv(N, tm2)

    s2, adjb = pl.pallas_call(
        functools.partial(_front_kernel, tm=tm),
        out_shape=(jax.ShapeDtypeStruct((N, enc2), _BF),
                   jax.ShapeDtypeStruct((N, N), _F8)),
        grid=(2, G),
        in_specs=[
            pl.BlockSpec((tm, n_input),
                         lambda p, k: (k * (p == 0) + last * (p == 1), 0)),
            pl.BlockSpec((tm, N),
                         lambda p, k: (k * (p == 1) + last * (p == 0), 0)),
            _full_spec((n_input, enc1)), _full_spec((enc1, enc2)),
        ],
        out_specs=(pl.BlockSpec((tm, enc2), lambda p, k: (k * (p == 1), 0)),
                   pl.BlockSpec((tm, N), lambda p, k: (k * (p == 1), 0))),
        scratch_shapes=[pltpu.VMEM((N, enc1), _BF)],
        compiler_params=pltpu.CompilerParams(
            dimension_semantics=("arbitrary", "arbitrary"),
            vmem_limit_bytes=_VMEM_LIMIT),
    )(x, adj, w1, w2)

    z_igae, c = pl.pallas_call(
        functools.partial(_back_kernel, tm=tm2),
        out_shape=(jax.ShapeDtypeStruct((N, enc3), jnp.float32),
                   jax.ShapeDtypeStruct((N, nc), jnp.float32)),
        grid=(2, G2),
        in_specs=[
            pl.BlockSpec((tm2, N), lambda p, k: (k, 0)),
            _full_spec((N, enc2)), _full_spec((enc2, enc3)),
            _full_spec((enc3, enc3)), _full_spec((1, enc3)),
            _full_spec((enc3, nc)), _full_spec((1, nc)),
        ],
        out_specs=(pl.BlockSpec((tm2, enc3), lambda p, k: (k * (p == 1), 0)),
                   pl.BlockSpec((tm2, nc), lambda p, k: (k * (p == 1), 0))),
        scratch_shapes=[pltpu.VMEM((N, enc3), _BF)],
        compiler_params=pltpu.CompilerParams(
            dimension_semantics=("arbitrary", "arbitrary"),
            vmem_limit_bytes=_VMEM_LIMIT),
    )(adjb, s2, w3, wc1, bc1, wc2, bc2)

    return z_igae, c


# speedup vs baseline: 1.7476x; 1.2065x over previous
import functools

import jax
import jax.numpy as jnp
from jax.experimental import pallas as pl
from jax.experimental.pallas import tpu as pltpu

_VMEM_LIMIT = 56 * 1024 * 1024
_BF = jnp.bfloat16
_F8 = jnp.float8_e4m3fn
_ADJ_SCALE = 256.0
_INV_ADJ_SCALE = 1.0 / 256.0


def _mega_kernel(x_ref, adj_ref, w1_ref, w2_ref, w3_ref, wc1_ref, bc1_ref,
                 wc2_ref, bc2_ref, z_ref, c_ref, s1_ref, s2_ref, s3_ref,
                 a8_ref, *, tma, ga, tmc, gc):
    i = pl.program_id(0)

    @pl.when(i < ga)
    def _phase_a():
        rows = pl.ds(i * tma, tma)
        s1 = jnp.dot(x_ref[...].astype(_BF), w1_ref[...].astype(_BF),
                     preferred_element_type=jnp.float32)
        s1_ref[rows, :] = jnp.tanh(s1).astype(_BF)

    @pl.when((i >= ga) & (i < 2 * ga))
    def _phase_b():
        rows = pl.ds((i - ga) * tma, tma)
        adjf = adj_ref[...]
        a8_ref[rows, :] = (adjf * _ADJ_SCALE).astype(_F8)
        z1 = jnp.dot(adjf.astype(_BF), s1_ref[...],
                     preferred_element_type=jnp.float32)
        s2 = jnp.dot(z1.astype(_BF), w2_ref[...].astype(_BF),
                     preferred_element_type=jnp.float32)
        s2_ref[rows, :] = jnp.tanh(s2).astype(_BF)

    @pl.when((i >= 2 * ga) & (i < 2 * ga + gc))
    def _phase_c():
        rows = pl.ds((i - 2 * ga) * tmc, tmc)
        adjb = a8_ref[rows, :].astype(_BF)
        z2 = jnp.dot(adjb, s2_ref[...], preferred_element_type=jnp.float32)
        w3 = (w3_ref[...] * _INV_ADJ_SCALE).astype(_BF)
        s3 = jnp.dot(z2.astype(_BF), w3, preferred_element_type=jnp.float32)
        s3_ref[rows, :] = s3.astype(_BF)

    @pl.when(i >= 2 * ga + gc)
    def _phase_d():
        rows = pl.ds((i - 2 * ga - gc) * tmc, tmc)
        adjb = a8_ref[rows, :].astype(_BF)
        z = jnp.dot(adjb, s3_ref[...],
                    preferred_element_type=jnp.float32) * _INV_ADJ_SCALE
        z_ref[...] = z
        h = jnp.dot(z.astype(_BF), wc1_ref[...].astype(_BF),
                    preferred_element_type=jnp.float32) + bc1_ref[...]
        h = jnp.maximum(h, 0.0)
        logits = jnp.dot(h.astype(_BF), wc2_ref[...].astype(_BF),
                         preferred_element_type=jnp.float32) + bc2_ref[...]
        m = jnp.max(logits, axis=-1, keepdims=True)
        e = jnp.exp(logits - m)
        c_ref[...] = e * pl.reciprocal(jnp.sum(e, axis=-1, keepdims=True))


def _full_spec(shape):
    return pl.BlockSpec(shape, lambda i, _s=shape: tuple(0 for _ in _s))


def kernel(x, adj, w1, w2, w3, wc1, bc1, wc2, bc2):
    N, n_input = x.shape
    enc1, enc2, enc3 = w1.shape[1], w2.shape[1], w3.shape[1]
    nc = wc2.shape[1]
    tma = min(512, N)
    ga = pl.cdiv(N, tma)
    tmc = min(2048, N)
    gc = pl.cdiv(N, tmc)
    grid = (2 * ga + 2 * gc,)

    z_igae, c = pl.pallas_call(
        functools.partial(_mega_kernel, tma=tma, ga=ga, tmc=tmc, gc=gc),
        out_shape=(jax.ShapeDtypeStruct((N, enc3), jnp.float32),
                   jax.ShapeDtypeStruct((N, nc), jnp.float32)),
        grid=grid,
        in_specs=[
            pl.BlockSpec((tma, n_input), lambda i: (i * (i < ga), 0)),
            pl.BlockSpec((tma, N),
                         lambda i: ((i - ga) * ((i >= ga) & (i < 2 * ga))
                                    + (ga - 1) * (i >= 2 * ga), 0)),
            _full_spec((n_input, enc1)), _full_spec((enc1, enc2)),
            _full_spec((enc2, enc3)), _full_spec((enc3, enc3)),
            _full_spec((1, enc3)), _full_spec((enc3, nc)),
            _full_spec((1, nc)),
        ],
        out_specs=(
            pl.BlockSpec((tmc, enc3),
                         lambda i: ((i - (2 * ga + gc)) * (i >= 2 * ga + gc),
                                    0)),
            pl.BlockSpec((tmc, nc),
                         lambda i: ((i - (2 * ga + gc)) * (i >= 2 * ga + gc),
                                    0)),
        ),
        scratch_shapes=[pltpu.VMEM((N, enc1), _BF),
                        pltpu.VMEM((N, enc2), _BF),
                        pltpu.VMEM((N, enc3), _BF),
                        pltpu.VMEM((N, N), _F8)],
        compiler_params=pltpu.CompilerParams(
            dimension_semantics=("arbitrary",),
            vmem_limit_bytes=_VMEM_LIMIT),
    )(x, adj, w1, w2, w3, wc1, bc1, wc2, bc2)

    return z_igae, c
